# baseline (speedup 1.0000x reference)
import functools

import jax
import jax.numpy as jnp
from jax import lax
from jax.experimental import pallas as pl
from jax.experimental.pallas import tpu as pltpu
from jax.experimental.pallas import tpu_sc as plsc

N_NODES = 10000
N_EDGES = 160000
IN_CH = 256
OUT_CH = 256

_NC = 2
_NS = 16
_L = 16
_HALF = OUT_CH // 2
_CH = 80
_CPT = N_EDGES // (_NS * _CH)
_RPT_BIG = 640
_RPT_LAST = N_NODES - (_NS - 1) * _RPT_BIG
_ZROWS = 80
_SUP = 5
_NSUP = _CPT // _SUP


def _mm_body(p_ref, x_ref, w_ref, o_ref):
    p = p_ref[0]
    acc = lax.dot_general(x_ref[...], w_ref[0],
                          (((1,), (1,)), ((), ())),
                          preferred_element_type=jnp.float32)
    o_ref[...] = (acc * p)[None]


def _project(ew_param, x, w2):
    bm = 2000
    return pl.pallas_call(
        _mm_body,
        grid=(2, N_NODES // bm),
        in_specs=[
            pl.BlockSpec(memory_space=pltpu.SMEM),
            pl.BlockSpec((bm, IN_CH), lambda h, i: (i, 0)),
            pl.BlockSpec((1, _HALF, IN_CH), lambda h, i: (h, 0, 0)),
        ],
        out_specs=pl.BlockSpec((1, bm, _HALF), lambda h, i: (h, i, 0)),
        out_shape=jax.ShapeDtypeStruct((2, N_NODES, _HALF), jnp.float32),
    )(ew_param, x, w2)


def _agg_body(xls_hbm, e_hbm, ew_hbm, out_hbm, acc, srcb, dstb, ewb, rows,
              srows, gsem, esem, ssem):
    c = lax.axis_index("c")
    s = lax.axis_index("s")

    def _zero_rows(i, carry):
        rows[0, i // 8, pl.ds((i % 8) * _L, _L)] = jnp.zeros((_L,), jnp.float32)
        return carry
    lax.fori_loop(0, _ZROWS * 8, _zero_rows, 0)

    r0 = s * _RPT_BIG
    nz = jnp.where(s < _NS - 1, _RPT_BIG // _ZROWS, _RPT_LAST // _ZROWS)

    def _zero_acc(k, carry):
        pltpu.sync_copy(rows.at[0], acc.at[pl.ds(r0 + k * _ZROWS, _ZROWS)])
        return carry
    lax.fori_loop(0, nz, _zero_acc, 0)
    plsc.subcore_barrier()

    pltpu.sync_copy(e_hbm.at[0, s, 0], srcb.at[0])
    pltpu.sync_copy(e_hbm.at[1, s, 0], dstb.at[0])
    pltpu.sync_copy(ew_hbm.at[s, 0], ewb.at[0])
    pltpu.async_copy(xls_hbm.at[c].at[srcb.at[0, 0]], rows.at[0], gsem.at[0])
    pltpu.async_copy(e_hbm.at[0, s, 1], srcb.at[1], esem)
    pltpu.async_copy(e_hbm.at[1, s, 1], dstb.at[1], esem)
    pltpu.async_copy(ew_hbm.at[s, 1], ewb.at[1], esem)

    def _iter(j, carry):
        b = j % 2
        b1 = 1 - b
        jj = j // _SUP
        q = j % _SUP
        slot = jj % 2

        @pl.when((q == _SUP - 1) & (j < _CPT - 1))
        def _():
            pltpu.make_async_copy(e_hbm.at[0, s, 0], srcb.at[0], esem).wait()
            pltpu.make_async_copy(e_hbm.at[0, s, 0], dstb.at[0], esem).wait()
            pltpu.make_async_copy(ew_hbm.at[s, 0], ewb.at[0], esem).wait()

        @pl.when(j < _CPT - 1)
        def _():
            j1 = j + 1
            pltpu.async_copy(
                xls_hbm.at[c].at[srcb.at[(j1 // _SUP) % 2, j1 % _SUP]],
                rows.at[b1], gsem.at[b1])


        pltpu.make_async_copy(xls_hbm.at[0, pl.ds(0, _CH)], rows.at[b],
                              gsem.at[b]).wait()

        @pl.when(j >= 2)
        def _():
            pltpu.make_async_copy(xls_hbm.at[0, pl.ds(0, _CH)], srows.at[b],
                                  ssem).wait()

        @pl.when((q == 1) & (jj >= 1) & (jj < _NSUP - 1))
        def _():
            pltpu.async_copy(e_hbm.at[0, s, jj + 1], srcb.at[1 - slot], esem)
            pltpu.async_copy(e_hbm.at[1, s, jj + 1], dstb.at[1 - slot], esem)
            pltpu.async_copy(ew_hbm.at[s, jj + 1], ewb.at[1 - slot], esem)

        def _scale(g, carry2):
            evw = ewb[slot, q, pl.ds(g * _L, _L)]
            base = g * _L
            for r16 in range(_L):
                sv = evw[r16]
                row = base + r16
                vals = [rows[b, row, pl.ds(k * _L, _L)]
                        for k in range(_HALF // _L)]
                for k in range(_HALF // _L):
                    srows[b, row, pl.ds(k * _L, _L)] = vals[k] * sv
            return carry2
        lax.fori_loop(0, _CH // _L, _scale, 0)

        pltpu.async_copy(srows.at[b], acc.at[dstb.at[slot, q]], ssem, add=True)
        return carry
    lax.fori_loop(0, _CPT, _iter, 0)

    pltpu.make_async_copy(xls_hbm.at[0, pl.ds(0, _CH)], srows.at[0], ssem).wait()
    pltpu.make_async_copy(xls_hbm.at[0, pl.ds(0, _CH)], srows.at[1], ssem).wait()
    plsc.subcore_barrier()

    for ci, c0 in ((0, 0), (1, _HALF)):
        @pl.when((c == ci) & (s < _NS - 1))
        def _(c0=c0):
            pltpu.sync_copy(acc.at[pl.ds(r0, _RPT_BIG)],
                            out_hbm.at[pl.ds(r0, _RPT_BIG), pl.ds(c0, _HALF)])

        @pl.when((c == ci) & (s == _NS - 1))
        def _(c0=c0):
            pltpu.sync_copy(acc.at[pl.ds(r0, _RPT_LAST)],
                            out_hbm.at[pl.ds(r0, _RPT_LAST), pl.ds(c0, _HALF)])


def _aggregate(xls, e, ew):
    mesh = plsc.VectorSubcoreMesh(core_axis_name="c", subcore_axis_name="s")
    return pl.kernel(
        _agg_body,
        out_type=jax.ShapeDtypeStruct((N_NODES, OUT_CH), jnp.float32),
        mesh=mesh,
        scratch_types=[
            pltpu.VMEM_SHARED((N_NODES, _HALF), jnp.float32),
            pltpu.VMEM((2, _SUP, _CH), jnp.int32),
            pltpu.VMEM((2, _SUP, _CH), jnp.int32),
            pltpu.VMEM((2, _SUP, _CH), jnp.float32),
            pltpu.VMEM((2, _CH, _HALF), jnp.float32),
            pltpu.VMEM((2, _CH, _HALF), jnp.float32),
            pltpu.SemaphoreType.DMA((2,)),
            pltpu.SemaphoreType.DMA,
            pltpu.SemaphoreType.DMA,
        ],
    )(xls, e, ew)


def kernel(x, edge_index, edge_weight, W, ew_param):
    w2 = W.reshape(2, _HALF, IN_CH)
    xls = _project(ew_param, x, w2)
    e = edge_index.astype(jnp.int32).reshape(2, _NS, _NSUP, _SUP, _CH)
    ew = edge_weight.reshape(_NS, _NSUP, _SUP, _CH)
    return _aggregate(xls, e, ew)


# speedup vs baseline: 1.0179x; 1.0179x over previous
import functools

import jax
import jax.numpy as jnp
from jax import lax
from jax.experimental import pallas as pl
from jax.experimental.pallas import tpu as pltpu
from jax.experimental.pallas import tpu_sc as plsc

N_NODES = 10000
N_EDGES = 160000
IN_CH = 256
OUT_CH = 256

_NC = 2
_NS = 16
_L = 16
_HALF = OUT_CH // 2
_CH = 80
_CPT = N_EDGES // (_NS * _CH)
_RPT_BIG = 640
_RPT_LAST = N_NODES - (_NS - 1) * _RPT_BIG
_ZROWS = 80


def _mm_body(p_ref, x_ref, w_ref, o_ref):
    p = p_ref[0]
    acc = lax.dot_general(x_ref[...], w_ref[0],
                          (((1,), (1,)), ((), ())),
                          preferred_element_type=jnp.float32)
    o_ref[...] = (acc * p)[None]


def _project(ew_param, x, w2):
    bm = 2000
    return pl.pallas_call(
        _mm_body,
        grid=(2, N_NODES // bm),
        in_specs=[
            pl.BlockSpec(memory_space=pltpu.SMEM),
            pl.BlockSpec((bm, IN_CH), lambda h, i: (i, 0)),
            pl.BlockSpec((1, _HALF, IN_CH), lambda h, i: (h, 0, 0)),
        ],
        out_specs=pl.BlockSpec((1, bm, _HALF), lambda h, i: (h, i, 0)),
        out_shape=jax.ShapeDtypeStruct((2, N_NODES, _HALF), jnp.float32),
    )(ew_param, x, w2)


def _agg_body(xls_hbm, e_hbm, ew_hbm, out_hbm, acc, srcb, dstb, ewb, rows,
              srows, gsem, esem, ssem):
    c = lax.axis_index("c")
    s = lax.axis_index("s")

    def _zero_rows(i, carry):
        rows[0, i // 8, pl.ds((i % 8) * _L, _L)] = jnp.zeros((_L,), jnp.float32)
        return carry
    lax.fori_loop(0, _ZROWS * 8, _zero_rows, 0)

    r0 = s * _RPT_BIG
    nz = jnp.where(s < _NS - 1, _RPT_BIG // _ZROWS, _RPT_LAST // _ZROWS)

    def _zero_acc(k, carry):
        pltpu.sync_copy(rows.at[0], acc.at[pl.ds(r0 + k * _ZROWS, _ZROWS)])
        return carry
    lax.fori_loop(0, nz, _zero_acc, 0)
    plsc.subcore_barrier()

    pltpu.sync_copy(e_hbm.at[0, s, 0], srcb.at[0])
    pltpu.sync_copy(e_hbm.at[1, s, 0], dstb.at[0])
    pltpu.sync_copy(ew_hbm.at[s, 0], ewb.at[0])
    pltpu.async_copy(xls_hbm.at[c].at[srcb.at[0]], rows.at[0], gsem.at[0])
    pltpu.async_copy(e_hbm.at[0, s, 1], srcb.at[1], esem)
    pltpu.async_copy(e_hbm.at[1, s, 1], dstb.at[1], esem)
    pltpu.async_copy(ew_hbm.at[s, 1], ewb.at[1], esem)

    def _iter(j, carry):
        b = j % 2
        b1 = 1 - b

        @pl.when(j < _CPT - 1)
        def _():
            pltpu.make_async_copy(e_hbm.at[0, s, 0], srcb.at[0], esem).wait()
            pltpu.make_async_copy(e_hbm.at[0, s, 0], dstb.at[0], esem).wait()
            pltpu.make_async_copy(ew_hbm.at[s, 0], ewb.at[0], esem).wait()

        @pl.when(j < _CPT - 1)
        def _():
            pltpu.async_copy(xls_hbm.at[c].at[srcb.at[(j + 1) % 3]],
                             rows.at[b1], gsem.at[b1])

        @pl.when(j < _CPT - 2)
        def _():
            pltpu.async_copy(e_hbm.at[0, s, j + 2], srcb.at[(j + 2) % 3], esem)
            pltpu.async_copy(e_hbm.at[1, s, j + 2], dstb.at[(j + 2) % 3], esem)
            pltpu.async_copy(ew_hbm.at[s, j + 2], ewb.at[(j + 2) % 3], esem)

        pltpu.make_async_copy(xls_hbm.at[0, pl.ds(0, _CH)], rows.at[b],
                              gsem.at[b]).wait()

        @pl.when(j >= 2)
        def _():
            pltpu.make_async_copy(xls_hbm.at[0, pl.ds(0, _CH)], srows.at[b],
                                  ssem).wait()
        eb = j % 3

        def _scale(g, carry2):
            evw = ewb[eb, pl.ds(g * _L, _L)]
            base = g * _L
            for r16 in range(_L):
                sv = evw[r16]
                row = base + r16
                vals = [rows[b, row, pl.ds(k * _L, _L)]
                        for k in range(_HALF // _L)]
                for k in range(_HALF // _L):
                    srows[b, row, pl.ds(k * _L, _L)] = vals[k] * sv
            return carry2
        lax.fori_loop(0, _CH // _L, _scale, 0)

        pltpu.async_copy(srows.at[b], acc.at[dstb.at[eb]], ssem, add=True)
        return carry
    lax.fori_loop(0, _CPT, _iter, 0)

    pltpu.make_async_copy(xls_hbm.at[0, pl.ds(0, _CH)], srows.at[0], ssem).wait()
    pltpu.make_async_copy(xls_hbm.at[0, pl.ds(0, _CH)], srows.at[1], ssem).wait()
    plsc.subcore_barrier()

    for ci, c0 in ((0, 0), (1, _HALF)):
        @pl.when((c == ci) & (s < _NS - 1))
        def _(c0=c0):
            pltpu.sync_copy(acc.at[pl.ds(r0, _RPT_BIG)],
                            out_hbm.at[pl.ds(r0, _RPT_BIG), pl.ds(c0, _HALF)])

        @pl.when((c == ci) & (s == _NS - 1))
        def _(c0=c0):
            pltpu.sync_copy(acc.at[pl.ds(r0, _RPT_LAST)],
                            out_hbm.at[pl.ds(r0, _RPT_LAST), pl.ds(c0, _HALF)])


def _aggregate(xls, e, ew):
    mesh = plsc.VectorSubcoreMesh(core_axis_name="c", subcore_axis_name="s")
    return pl.kernel(
        _agg_body,
        out_type=jax.ShapeDtypeStruct((N_NODES, OUT_CH), jnp.float32),
        mesh=mesh,
        scratch_types=[
            pltpu.VMEM_SHARED((N_NODES, _HALF), jnp.float32),
            pltpu.VMEM((3, _CH), jnp.int32),
            pltpu.VMEM((3, _CH), jnp.int32),
            pltpu.VMEM((3, _CH), jnp.float32),
            pltpu.VMEM((2, _CH, _HALF), jnp.float32),
            pltpu.VMEM((2, _CH, _HALF), jnp.float32),
            pltpu.SemaphoreType.DMA((2,)),
            pltpu.SemaphoreType.DMA,
            pltpu.SemaphoreType.DMA,
        ],
    )(xls, e, ew)


def kernel(x, edge_index, edge_weight, W, ew_param):
    w2 = W.reshape(2, _HALF, IN_CH)
    xls = _project(ew_param, x, w2)
    e = edge_index.astype(jnp.int32).reshape(2, _NS, _CPT, _CH)
    ew = edge_weight.reshape(_NS, _CPT, _CH)
    return _aggregate(xls, e, ew)
